# baseline (speedup 1.0000x reference)
import functools

import jax
import jax.numpy as jnp
from jax import lax
from jax.experimental import pallas as pl
from jax.experimental.pallas import tpu as pltpu
from jax.experimental.pallas import tpu_sc as plsc

N_NODES = 10000
C = 128
NUM_LAYERS = 3

_NC = 2
_NS = 16
_NW = _NC * _NS
_CHUNK = 128
_Q0 = 160
_Q1 = 0
_TOTAL_CHUNKS = _NS * (_Q0 + _Q1)
_EPAD = _TOTAL_CHUNKS * _CHUNK
_ROWS_PER_TILE = 640
_ACC_ROWS = _NS * _ROWS_PER_TILE

_BN = 1000
_GRID = N_NODES // _BN


_NBUF = 2
_SEG = 16
_MAX_NSEG = _Q0 // _SEG
_SEG_ROUNDS = _SEG // _NBUF


@functools.cache
def _sc_edge_pass_kernel():
    mesh = plsc.VectorSubcoreMesh(core_axis_name="c", subcore_axis_name="s")

    @functools.partial(
        pl.kernel,
        out_type=jax.ShapeDtypeStruct((_NC * _ACC_ROWS, C), jnp.float32),
        mesh=mesh,
        scratch_types=[
            pltpu.VMEM((_SEG, _CHUNK), jnp.int32),
            pltpu.VMEM((_SEG, _CHUNK), jnp.int32),
        ]
        + [pltpu.VMEM((_CHUNK, C), jnp.float32) for _ in range(_NBUF)]
        + [pltpu.VMEM_SHARED((_ACC_ROWS, C), jnp.float32)]
        + [pltpu.SemaphoreType.DMA for _ in range(_NBUF)],
    )
    def sc_edge_pass(m_hbm, src_hbm, dst_hbm, zeros_hbm, out_hbm,
                     src_v, dst_v, *rest):
        msgs = rest[:_NBUF]
        acc_sh = rest[_NBUF]
        sems = rest[_NBUF + 1:]
        c = lax.axis_index("c")
        s = lax.axis_index("s")
        q = jnp.where(c == 0, _Q0, _Q1)
        base = jnp.where(c == 0, s * _Q0, _NS * _Q0 + s * _Q1)
        pltpu.sync_copy(zeros_hbm, acc_sh.at[pl.ds(s * _ROWS_PER_TILE, _ROWS_PER_TILE)])
        plsc.subcore_barrier()

        for seg in range(_MAX_NSEG):

            @pl.when(seg * _SEG < q)
            def _():
                sb = base + seg * _SEG
                pltpu.sync_copy(src_hbm.at[pl.ds(sb, _SEG)], src_v)
                pltpu.sync_copy(dst_hbm.at[pl.ds(sb, _SEG)], dst_v)
                for b in range(_NBUF):
                    pltpu.async_copy(m_hbm.at[src_v.at[b]], msgs[b], sems[b])

                def round_body(r, carry):
                    for b in range(_NBUF):
                        j = r * _NBUF + b
                        pltpu.make_async_copy(m_hbm.at[src_v.at[j]], msgs[b], sems[b]).wait()
                        pltpu.sync_copy(msgs[b], acc_sh.at[dst_v.at[j]], add=True)
                        pltpu.async_copy(m_hbm.at[src_v.at[j + _NBUF]], msgs[b], sems[b])
                    return carry

                lax.fori_loop(0, _SEG_ROUNDS - 1, round_body, 0)
                for b in range(_NBUF):
                    j = (_SEG_ROUNDS - 1) * _NBUF + b
                    pltpu.make_async_copy(m_hbm.at[src_v.at[j]], msgs[b], sems[b]).wait()
                    pltpu.sync_copy(msgs[b], acc_sh.at[dst_v.at[j]], add=True)

        plsc.subcore_barrier()
        off = c * _ACC_ROWS + s * _ROWS_PER_TILE
        pltpu.sync_copy(acc_sh.at[pl.ds(s * _ROWS_PER_TILE, _ROWS_PER_TILE)],
                        out_hbm.at[pl.ds(off, _ROWS_PER_TILE)])

    return sc_edge_pass


def _sc_edge_pass(m, src_slabs, dst_slabs, zeros):
    return _sc_edge_pass_kernel()(m, src_slabs, dst_slabs, zeros)


def _mm_body(x_ref, w_ref, o_ref):
    o_ref[...] = jnp.dot(x_ref[...], w_ref[...],
                         preferred_element_type=jnp.float32)


def _mm(x, w):
    return pl.pallas_call(
        _mm_body,
        grid=(_GRID,),
        in_specs=[pl.BlockSpec((_BN, C), lambda i: (i, 0)),
                  pl.BlockSpec((C, C), lambda i: (0, 0))],
        out_specs=pl.BlockSpec((_BN, C), lambda i: (i, 0)),
        out_shape=jax.ShapeDtypeStruct((N_NODES, C), jnp.float32),
    )(x, w)


def _gru_math(p0_ref, p1_ref, x_ref, wih_ref, whh_ref, b_ref):
    agg = p0_ref[0] + p1_ref[0]
    h = x_ref[...]
    gi = jnp.dot(agg, wih_ref[...], preferred_element_type=jnp.float32) + b_ref[0:1, :]
    gh = jnp.dot(h, whh_ref[...], preferred_element_type=jnp.float32) + b_ref[1:2, :]
    r = jax.nn.sigmoid(gi[:, :C] + gh[:, :C])
    z = jax.nn.sigmoid(gi[:, C:2 * C] + gh[:, C:2 * C])
    n = jnp.tanh(gi[:, 2 * C:] + r * gh[:, 2 * C:])
    return (1.0 - z) * n + z * h


def _gru_fused_body(p0_ref, p1_ref, x_ref, wih_ref, whh_ref, b_ref, wn_ref,
                    x_out_ref, m_out_ref):
    xn = _gru_math(p0_ref, p1_ref, x_ref, wih_ref, whh_ref, b_ref)
    x_out_ref[...] = xn
    m_out_ref[...] = jnp.dot(xn, wn_ref[...], preferred_element_type=jnp.float32)


def _gru_last_body(p0_ref, p1_ref, x_ref, wih_ref, whh_ref, b_ref, x_out_ref):
    x_out_ref[...] = _gru_math(p0_ref, p1_ref, x_ref, wih_ref, whh_ref, b_ref)


def _gru_specs():
    return [
        pl.BlockSpec((1, _BN, C), lambda i: (0, i, 0)),
        pl.BlockSpec((1, _BN, C), lambda i: (1, i, 0)),
        pl.BlockSpec((_BN, C), lambda i: (i, 0)),
        pl.BlockSpec((C, 3 * C), lambda i: (0, 0)),
        pl.BlockSpec((C, 3 * C), lambda i: (0, 0)),
        pl.BlockSpec((8, 3 * C), lambda i: (0, 0)),
    ]


def _gru_fused(parts, x, wih_t, whh_t, bias, w_next):
    return pl.pallas_call(
        _gru_fused_body,
        grid=(_GRID,),
        in_specs=_gru_specs() + [pl.BlockSpec((C, C), lambda i: (0, 0))],
        out_specs=[pl.BlockSpec((_BN, C), lambda i: (i, 0)),
                   pl.BlockSpec((_BN, C), lambda i: (i, 0))],
        out_shape=[jax.ShapeDtypeStruct((N_NODES, C), jnp.float32),
                   jax.ShapeDtypeStruct((N_NODES, C), jnp.float32)],
    )(parts, parts, x, wih_t, whh_t, bias, w_next)


def _gru_last(parts, x, wih_t, whh_t, bias):
    return pl.pallas_call(
        _gru_last_body,
        grid=(_GRID,),
        in_specs=_gru_specs(),
        out_specs=pl.BlockSpec((_BN, C), lambda i: (i, 0)),
        out_shape=jax.ShapeDtypeStruct((N_NODES, C), jnp.float32),
    )(parts, parts, x, wih_t, whh_t, bias)


def kernel(x, edge_index, weight, w_ih, w_hh, b_ih, b_hh):
    src = edge_index[0].astype(jnp.int32)
    dst = edge_index[1].astype(jnp.int32)
    n_edges = src.shape[0]
    pad = _EPAD - n_edges
    src_slabs = jnp.concatenate(
        [src, jnp.zeros((pad,), jnp.int32)]).reshape(_TOTAL_CHUNKS, _CHUNK)
    dst_slabs = jnp.concatenate(
        [dst, jnp.full((pad,), N_NODES, jnp.int32)]).reshape(_TOTAL_CHUNKS, _CHUNK)
    zeros = jnp.zeros((_ROWS_PER_TILE, C), jnp.float32)
    wih_t = w_ih.T
    whh_t = w_hh.T
    bias = jnp.zeros((8, 3 * C), jnp.float32).at[0].set(b_ih).at[1].set(b_hh)

    m = _mm(x, weight[0])
    for i in range(NUM_LAYERS):
        parts = _sc_edge_pass(m, src_slabs, dst_slabs, zeros)
        parts = parts.reshape(_NC, _ACC_ROWS, C)
        if i + 1 < NUM_LAYERS:
            x, m = _gru_fused(parts, x, wih_t, whh_t, bias, weight[i + 1])
        else:
            x = _gru_last(parts, x, wih_t, whh_t, bias)
    return x


# speedup vs baseline: 1.3883x; 1.3883x over previous
import functools

import jax
import jax.numpy as jnp
from jax import lax
from jax.experimental import pallas as pl
from jax.experimental.pallas import tpu as pltpu
from jax.experimental.pallas import tpu_sc as plsc

N_NODES = 10000
C = 128
NUM_LAYERS = 3

_NC = 2
_NS = 16
_NW = _NC * _NS
_CHUNK = 128
_Q0 = 144
_Q1 = 16
_TOTAL_CHUNKS = _NS * (_Q0 + _Q1)
_EPAD = _TOTAL_CHUNKS * _CHUNK
_ROWS_PER_TILE = 640
_ACC_ROWS = _NS * _ROWS_PER_TILE

_BN = 1000
_GRID = N_NODES // _BN


_NBUF = 2
_SEG = 16
_MAX_NSEG = _Q0 // _SEG
_SEG_ROUNDS = _SEG // _NBUF


@functools.cache
def _sc_edge_pass_kernel():
    mesh = plsc.VectorSubcoreMesh(core_axis_name="c", subcore_axis_name="s")

    @functools.partial(
        pl.kernel,
        out_type=jax.ShapeDtypeStruct((_NC * _ACC_ROWS, C), jnp.float32),
        mesh=mesh,
        scratch_types=[
            pltpu.VMEM((_SEG, _CHUNK), jnp.int32),
            pltpu.VMEM((_SEG, _CHUNK), jnp.int32),
        ]
        + [pltpu.VMEM((_CHUNK, C), jnp.float32) for _ in range(_NBUF)]
        + [pltpu.VMEM_SHARED((_ACC_ROWS, C), jnp.float32)]
        + [pltpu.SemaphoreType.DMA for _ in range(_NBUF)],
    )
    def sc_edge_pass(m_hbm, src_hbm, dst_hbm, zeros_hbm, out_hbm,
                     src_v, dst_v, *rest):
        msgs = rest[:_NBUF]
        acc_sh = rest[_NBUF]
        sems = rest[_NBUF + 1:]
        c = lax.axis_index("c")
        s = lax.axis_index("s")
        q = jnp.where(c == 0, _Q0, _Q1)
        base = jnp.where(c == 0, s * _Q0, _NS * _Q0 + s * _Q1)
        pltpu.sync_copy(zeros_hbm, acc_sh.at[pl.ds(s * _ROWS_PER_TILE, _ROWS_PER_TILE)])
        plsc.subcore_barrier()

        for seg in range(_MAX_NSEG):

            @pl.when(seg * _SEG < q)
            def _():
                sb = base + seg * _SEG
                pltpu.sync_copy(src_hbm.at[pl.ds(sb, _SEG)], src_v)
                pltpu.sync_copy(dst_hbm.at[pl.ds(sb, _SEG)], dst_v)
                for b in range(_NBUF):
                    pltpu.async_copy(m_hbm.at[src_v.at[b]], msgs[b], sems[b])

                def round_body(r, carry):
                    for b in range(_NBUF):
                        j = r * _NBUF + b
                        pltpu.make_async_copy(m_hbm.at[src_v.at[j]], msgs[b], sems[b]).wait()
                        pltpu.sync_copy(msgs[b], acc_sh.at[dst_v.at[j]], add=True)
                        pltpu.async_copy(m_hbm.at[src_v.at[j + _NBUF]], msgs[b], sems[b])
                    return carry

                lax.fori_loop(0, _SEG_ROUNDS - 1, round_body, 0)
                for b in range(_NBUF):
                    j = (_SEG_ROUNDS - 1) * _NBUF + b
                    pltpu.make_async_copy(m_hbm.at[src_v.at[j]], msgs[b], sems[b]).wait()
                    pltpu.sync_copy(msgs[b], acc_sh.at[dst_v.at[j]], add=True)

        plsc.subcore_barrier()
        off = c * _ACC_ROWS + s * _ROWS_PER_TILE
        pltpu.sync_copy(acc_sh.at[pl.ds(s * _ROWS_PER_TILE, _ROWS_PER_TILE)],
                        out_hbm.at[pl.ds(off, _ROWS_PER_TILE)])

    return sc_edge_pass


def _sc_edge_pass(m, src_slabs, dst_slabs, zeros):
    return _sc_edge_pass_kernel()(m, src_slabs, dst_slabs, zeros)


def _mm_body(x_ref, w_ref, o_ref):
    o_ref[...] = jnp.dot(x_ref[...], w_ref[...],
                         preferred_element_type=jnp.float32)


def _mm(x, w):
    return pl.pallas_call(
        _mm_body,
        grid=(_GRID,),
        in_specs=[pl.BlockSpec((_BN, C), lambda i: (i, 0)),
                  pl.BlockSpec((C, C), lambda i: (0, 0))],
        out_specs=pl.BlockSpec((_BN, C), lambda i: (i, 0)),
        out_shape=jax.ShapeDtypeStruct((N_NODES, C), jnp.float32),
    )(x, w)


def _gru_math(p0_ref, p1_ref, x_ref, wih_ref, whh_ref, b_ref):
    agg = p0_ref[0] + p1_ref[0]
    h = x_ref[...]
    gi = jnp.dot(agg, wih_ref[...], preferred_element_type=jnp.float32) + b_ref[0:1, :]
    gh = jnp.dot(h, whh_ref[...], preferred_element_type=jnp.float32) + b_ref[1:2, :]
    r = jax.nn.sigmoid(gi[:, :C] + gh[:, :C])
    z = jax.nn.sigmoid(gi[:, C:2 * C] + gh[:, C:2 * C])
    n = jnp.tanh(gi[:, 2 * C:] + r * gh[:, 2 * C:])
    return (1.0 - z) * n + z * h


def _gru_fused_body(p0_ref, p1_ref, x_ref, wih_ref, whh_ref, b_ref, wn_ref,
                    x_out_ref, m_out_ref):
    xn = _gru_math(p0_ref, p1_ref, x_ref, wih_ref, whh_ref, b_ref)
    x_out_ref[...] = xn
    m_out_ref[...] = jnp.dot(xn, wn_ref[...], preferred_element_type=jnp.float32)


def _gru_last_body(p0_ref, p1_ref, x_ref, wih_ref, whh_ref, b_ref, x_out_ref):
    x_out_ref[...] = _gru_math(p0_ref, p1_ref, x_ref, wih_ref, whh_ref, b_ref)


def _gru_specs():
    return [
        pl.BlockSpec((1, _BN, C), lambda i: (0, i, 0)),
        pl.BlockSpec((1, _BN, C), lambda i: (1, i, 0)),
        pl.BlockSpec((_BN, C), lambda i: (i, 0)),
        pl.BlockSpec((C, 3 * C), lambda i: (0, 0)),
        pl.BlockSpec((C, 3 * C), lambda i: (0, 0)),
        pl.BlockSpec((8, 3 * C), lambda i: (0, 0)),
    ]


def _gru_fused(parts, x, wih_t, whh_t, bias, w_next):
    return pl.pallas_call(
        _gru_fused_body,
        grid=(_GRID,),
        in_specs=_gru_specs() + [pl.BlockSpec((C, C), lambda i: (0, 0))],
        out_specs=[pl.BlockSpec((_BN, C), lambda i: (i, 0)),
                   pl.BlockSpec((_BN, C), lambda i: (i, 0))],
        out_shape=[jax.ShapeDtypeStruct((N_NODES, C), jnp.float32),
                   jax.ShapeDtypeStruct((N_NODES, C), jnp.float32)],
    )(parts, parts, x, wih_t, whh_t, bias, w_next)


def _gru_last(parts, x, wih_t, whh_t, bias):
    return pl.pallas_call(
        _gru_last_body,
        grid=(_GRID,),
        in_specs=_gru_specs(),
        out_specs=pl.BlockSpec((_BN, C), lambda i: (i, 0)),
        out_shape=jax.ShapeDtypeStruct((N_NODES, C), jnp.float32),
    )(parts, parts, x, wih_t, whh_t, bias)


def kernel(x, edge_index, weight, w_ih, w_hh, b_ih, b_hh):
    src = edge_index[0].astype(jnp.int32)
    dst = edge_index[1].astype(jnp.int32)
    n_edges = src.shape[0]
    pad = _EPAD - n_edges
    src_slabs = jnp.concatenate(
        [src, jnp.zeros((pad,), jnp.int32)]).reshape(_TOTAL_CHUNKS, _CHUNK)
    dst_slabs = jnp.concatenate(
        [dst, jnp.full((pad,), N_NODES, jnp.int32)]).reshape(_TOTAL_CHUNKS, _CHUNK)
    zeros = jnp.zeros((_ROWS_PER_TILE, C), jnp.float32)
    wih_t = w_ih.T
    whh_t = w_hh.T
    bias = jnp.zeros((8, 3 * C), jnp.float32).at[0].set(b_ih).at[1].set(b_hh)

    m = _mm(x, weight[0])
    for i in range(NUM_LAYERS):
        parts = _sc_edge_pass(m, src_slabs, dst_slabs, zeros)
        parts = parts.reshape(_NC, _ACC_ROWS, C)
        if i + 1 < NUM_LAYERS:
            x, m = _gru_fused(parts, x, wih_t, whh_t, bias, weight[i + 1])
        else:
            x = _gru_last(parts, x, wih_t, whh_t, bias)
    return x


# speedup vs baseline: 1.3929x; 1.0033x over previous
import functools

import jax
import jax.numpy as jnp
from jax import lax
from jax.experimental import pallas as pl
from jax.experimental.pallas import tpu as pltpu
from jax.experimental.pallas import tpu_sc as plsc

N_NODES = 10000
C = 128
NUM_LAYERS = 3

_NC = 2
_NS = 16
_NW = _NC * _NS
_CHUNK = 128
_Q0 = 152
_Q1 = 8
_TOTAL_CHUNKS = _NS * (_Q0 + _Q1)
_EPAD = _TOTAL_CHUNKS * _CHUNK
_ROWS_PER_TILE = 640
_ACC_ROWS = _NS * _ROWS_PER_TILE

_BN = 1000
_GRID = N_NODES // _BN


_NBUF = 2
_SEG = 8
_MAX_NSEG = _Q0 // _SEG
_SEG_ROUNDS = _SEG // _NBUF


@functools.cache
def _sc_edge_pass_kernel():
    mesh = plsc.VectorSubcoreMesh(core_axis_name="c", subcore_axis_name="s")

    @functools.partial(
        pl.kernel,
        out_type=jax.ShapeDtypeStruct((_NC * _ACC_ROWS, C), jnp.float32),
        mesh=mesh,
        scratch_types=[
            pltpu.VMEM((_SEG, _CHUNK), jnp.int32),
            pltpu.VMEM((_SEG, _CHUNK), jnp.int32),
        ]
        + [pltpu.VMEM((_CHUNK, C), jnp.float32) for _ in range(_NBUF)]
        + [pltpu.VMEM_SHARED((_ACC_ROWS, C), jnp.float32)]
        + [pltpu.SemaphoreType.DMA for _ in range(_NBUF)],
    )
    def sc_edge_pass(m_hbm, src_hbm, dst_hbm, zeros_hbm, out_hbm,
                     src_v, dst_v, *rest):
        msgs = rest[:_NBUF]
        acc_sh = rest[_NBUF]
        sems = rest[_NBUF + 1:]
        c = lax.axis_index("c")
        s = lax.axis_index("s")
        q = jnp.where(c == 0, _Q0, _Q1)
        base = jnp.where(c == 0, s * _Q0, _NS * _Q0 + s * _Q1)
        pltpu.sync_copy(zeros_hbm, acc_sh.at[pl.ds(s * _ROWS_PER_TILE, _ROWS_PER_TILE)])
        plsc.subcore_barrier()

        for seg in range(_MAX_NSEG):

            @pl.when(seg * _SEG < q)
            def _():
                sb = base + seg * _SEG
                pltpu.sync_copy(src_hbm.at[pl.ds(sb, _SEG)], src_v)
                pltpu.sync_copy(dst_hbm.at[pl.ds(sb, _SEG)], dst_v)
                for b in range(_NBUF):
                    pltpu.async_copy(m_hbm.at[src_v.at[b]], msgs[b], sems[b])

                def round_body(r, carry):
                    for b in range(_NBUF):
                        j = r * _NBUF + b
                        pltpu.make_async_copy(m_hbm.at[src_v.at[j]], msgs[b], sems[b]).wait()
                        pltpu.sync_copy(msgs[b], acc_sh.at[dst_v.at[j]], add=True)
                        pltpu.async_copy(m_hbm.at[src_v.at[j + _NBUF]], msgs[b], sems[b])
                    return carry

                lax.fori_loop(0, _SEG_ROUNDS - 1, round_body, 0)
                for b in range(_NBUF):
                    j = (_SEG_ROUNDS - 1) * _NBUF + b
                    pltpu.make_async_copy(m_hbm.at[src_v.at[j]], msgs[b], sems[b]).wait()
                    pltpu.sync_copy(msgs[b], acc_sh.at[dst_v.at[j]], add=True)

        plsc.subcore_barrier()
        off = c * _ACC_ROWS + s * _ROWS_PER_TILE
        pltpu.sync_copy(acc_sh.at[pl.ds(s * _ROWS_PER_TILE, _ROWS_PER_TILE)],
                        out_hbm.at[pl.ds(off, _ROWS_PER_TILE)])

    return sc_edge_pass


def _sc_edge_pass(m, src_slabs, dst_slabs, zeros):
    return _sc_edge_pass_kernel()(m, src_slabs, dst_slabs, zeros)


def _mm_body(x_ref, w_ref, o_ref):
    o_ref[...] = jnp.dot(x_ref[...], w_ref[...],
                         preferred_element_type=jnp.float32)


def _mm(x, w):
    return pl.pallas_call(
        _mm_body,
        grid=(_GRID,),
        in_specs=[pl.BlockSpec((_BN, C), lambda i: (i, 0)),
                  pl.BlockSpec((C, C), lambda i: (0, 0))],
        out_specs=pl.BlockSpec((_BN, C), lambda i: (i, 0)),
        out_shape=jax.ShapeDtypeStruct((N_NODES, C), jnp.float32),
    )(x, w)


def _gru_math(p0_ref, p1_ref, x_ref, wih_ref, whh_ref, b_ref):
    agg = p0_ref[0] + p1_ref[0]
    h = x_ref[...]
    gi = jnp.dot(agg, wih_ref[...], preferred_element_type=jnp.float32) + b_ref[0:1, :]
    gh = jnp.dot(h, whh_ref[...], preferred_element_type=jnp.float32) + b_ref[1:2, :]
    r = jax.nn.sigmoid(gi[:, :C] + gh[:, :C])
    z = jax.nn.sigmoid(gi[:, C:2 * C] + gh[:, C:2 * C])
    n = jnp.tanh(gi[:, 2 * C:] + r * gh[:, 2 * C:])
    return (1.0 - z) * n + z * h


def _gru_fused_body(p0_ref, p1_ref, x_ref, wih_ref, whh_ref, b_ref, wn_ref,
                    x_out_ref, m_out_ref):
    xn = _gru_math(p0_ref, p1_ref, x_ref, wih_ref, whh_ref, b_ref)
    x_out_ref[...] = xn
    m_out_ref[...] = jnp.dot(xn, wn_ref[...], preferred_element_type=jnp.float32)


def _gru_last_body(p0_ref, p1_ref, x_ref, wih_ref, whh_ref, b_ref, x_out_ref):
    x_out_ref[...] = _gru_math(p0_ref, p1_ref, x_ref, wih_ref, whh_ref, b_ref)


def _gru_specs():
    return [
        pl.BlockSpec((1, _BN, C), lambda i: (0, i, 0)),
        pl.BlockSpec((1, _BN, C), lambda i: (1, i, 0)),
        pl.BlockSpec((_BN, C), lambda i: (i, 0)),
        pl.BlockSpec((C, 3 * C), lambda i: (0, 0)),
        pl.BlockSpec((C, 3 * C), lambda i: (0, 0)),
        pl.BlockSpec((8, 3 * C), lambda i: (0, 0)),
    ]


def _gru_fused(parts, x, wih_t, whh_t, bias, w_next):
    return pl.pallas_call(
        _gru_fused_body,
        grid=(_GRID,),
        in_specs=_gru_specs() + [pl.BlockSpec((C, C), lambda i: (0, 0))],
        out_specs=[pl.BlockSpec((_BN, C), lambda i: (i, 0)),
                   pl.BlockSpec((_BN, C), lambda i: (i, 0))],
        out_shape=[jax.ShapeDtypeStruct((N_NODES, C), jnp.float32),
                   jax.ShapeDtypeStruct((N_NODES, C), jnp.float32)],
    )(parts, parts, x, wih_t, whh_t, bias, w_next)


def _gru_last(parts, x, wih_t, whh_t, bias):
    return pl.pallas_call(
        _gru_last_body,
        grid=(_GRID,),
        in_specs=_gru_specs(),
        out_specs=pl.BlockSpec((_BN, C), lambda i: (i, 0)),
        out_shape=jax.ShapeDtypeStruct((N_NODES, C), jnp.float32),
    )(parts, parts, x, wih_t, whh_t, bias)


def kernel(x, edge_index, weight, w_ih, w_hh, b_ih, b_hh):
    src = edge_index[0].astype(jnp.int32)
    dst = edge_index[1].astype(jnp.int32)
    n_edges = src.shape[0]
    pad = _EPAD - n_edges
    src_slabs = jnp.concatenate(
        [src, jnp.zeros((pad,), jnp.int32)]).reshape(_TOTAL_CHUNKS, _CHUNK)
    dst_slabs = jnp.concatenate(
        [dst, jnp.full((pad,), N_NODES, jnp.int32)]).reshape(_TOTAL_CHUNKS, _CHUNK)
    zeros = jnp.zeros((_ROWS_PER_TILE, C), jnp.float32)
    wih_t = w_ih.T
    whh_t = w_hh.T
    bias = jnp.zeros((8, 3 * C), jnp.float32).at[0].set(b_ih).at[1].set(b_hh)

    m = _mm(x, weight[0])
    for i in range(NUM_LAYERS):
        parts = _sc_edge_pass(m, src_slabs, dst_slabs, zeros)
        parts = parts.reshape(_NC, _ACC_ROWS, C)
        if i + 1 < NUM_LAYERS:
            x, m = _gru_fused(parts, x, wih_t, whh_t, bias, weight[i + 1])
        else:
            x = _gru_last(parts, x, wih_t, whh_t, bias)
    return x


# speedup vs baseline: 2.7677x; 1.9870x over previous
import functools

import jax
import jax.numpy as jnp
from jax import lax
from jax.experimental import pallas as pl
from jax.experimental.pallas import tpu as pltpu
from jax.experimental.pallas import tpu_sc as plsc

N_NODES = 10000
C = 128
NUM_LAYERS = 3

_NC = 2
_NS = 16
_CHUNK = 128
_HC = C // _NC
_TOTAL_CHUNKS = 2560
_CHUNKS_PER_TILE = _TOTAL_CHUNKS // _NS
_EPAD = _TOTAL_CHUNKS * _CHUNK
_M_ROWS_PER_TILE = 625
_ROWS_PER_TILE = 640
_ACC_ROWS = _NS * _ROWS_PER_TILE

_BN = 1000
_GRID = N_NODES // _BN


_NBUF = 4
_SEG = 16
_NSEG = _CHUNKS_PER_TILE // _SEG
_SEG_ROUNDS = _SEG // _NBUF


@functools.cache
def _sc_edge_pass_kernel():
    mesh = plsc.VectorSubcoreMesh(core_axis_name="c", subcore_axis_name="s")

    @functools.partial(
        pl.kernel,
        out_type=jax.ShapeDtypeStruct((_ACC_ROWS, C), jnp.float32),
        mesh=mesh,
        compiler_params=pltpu.CompilerParams(use_tc_tiling_on_sc=False),
        scratch_types=[
            pltpu.VMEM((_SEG, _CHUNK), jnp.int32),
            pltpu.VMEM((_SEG, _CHUNK), jnp.int32),
        ]
        + [pltpu.VMEM((_CHUNK, _HC), jnp.float32) for _ in range(_NBUF)]
        + [pltpu.VMEM_SHARED((N_NODES, _HC), jnp.float32)]
        + [pltpu.VMEM_SHARED((_ACC_ROWS, _HC), jnp.float32)]
        + [pltpu.SemaphoreType.DMA for _ in range(_NBUF)],
    )
    def sc_edge_pass(m_hbm, src_hbm, dst_hbm, zeros_hbm, out_hbm,
                     src_v, dst_v, *rest):
        msgs = rest[:_NBUF]
        m_sh = rest[_NBUF]
        acc_sh = rest[_NBUF + 1]
        sems = rest[_NBUF + 2:]
        c = lax.axis_index("c")
        s = lax.axis_index("s")
        pltpu.sync_copy(
            m_hbm.at[pl.ds(s * _M_ROWS_PER_TILE, _M_ROWS_PER_TILE),
                     pl.ds(c * _HC, _HC)],
            m_sh.at[pl.ds(s * _M_ROWS_PER_TILE, _M_ROWS_PER_TILE)])
        pltpu.sync_copy(zeros_hbm, acc_sh.at[pl.ds(s * _ROWS_PER_TILE, _ROWS_PER_TILE)])
        plsc.subcore_barrier()

        base = s * _CHUNKS_PER_TILE
        for seg in range(_NSEG):
            sb = base + seg * _SEG
            pltpu.sync_copy(src_hbm.at[pl.ds(sb, _SEG)], src_v)
            pltpu.sync_copy(dst_hbm.at[pl.ds(sb, _SEG)], dst_v)
            for b in range(_NBUF):
                pltpu.async_copy(m_sh.at[src_v.at[b]], msgs[b], sems[b])

            def round_body(r, carry):
                for b in range(_NBUF):
                    j = r * _NBUF + b
                    pltpu.make_async_copy(m_sh.at[src_v.at[j]], msgs[b], sems[b]).wait()
                    pltpu.sync_copy(msgs[b], acc_sh.at[dst_v.at[j]], add=True)
                    pltpu.async_copy(m_sh.at[src_v.at[j + _NBUF]], msgs[b], sems[b])
                return carry

            lax.fori_loop(0, _SEG_ROUNDS - 1, round_body, 0)
            for b in range(_NBUF):
                j = (_SEG_ROUNDS - 1) * _NBUF + b
                pltpu.make_async_copy(m_sh.at[src_v.at[j]], msgs[b], sems[b]).wait()
                pltpu.sync_copy(msgs[b], acc_sh.at[dst_v.at[j]], add=True)

        plsc.subcore_barrier()
        pltpu.sync_copy(acc_sh.at[pl.ds(s * _ROWS_PER_TILE, _ROWS_PER_TILE)],
                        out_hbm.at[pl.ds(s * _ROWS_PER_TILE, _ROWS_PER_TILE),
                                   pl.ds(c * _HC, _HC)])

    return sc_edge_pass


def _sc_edge_pass(m, src_slabs, dst_slabs, zeros):
    return _sc_edge_pass_kernel()(m, src_slabs, dst_slabs, zeros)


def _mm_body(x_ref, w_ref, o_ref):
    o_ref[...] = jnp.dot(x_ref[...], w_ref[...],
                         preferred_element_type=jnp.float32)


def _mm(x, w):
    return pl.pallas_call(
        _mm_body,
        grid=(_GRID,),
        in_specs=[pl.BlockSpec((_BN, C), lambda i: (i, 0)),
                  pl.BlockSpec((C, C), lambda i: (0, 0))],
        out_specs=pl.BlockSpec((_BN, C), lambda i: (i, 0)),
        out_shape=jax.ShapeDtypeStruct((N_NODES, C), jnp.float32),
    )(x, w)


def _gru_math(p_ref, x_ref, wih_ref, whh_ref, b_ref):
    agg = p_ref[...]
    h = x_ref[...]
    gi = jnp.dot(agg, wih_ref[...], preferred_element_type=jnp.float32) + b_ref[0:1, :]
    gh = jnp.dot(h, whh_ref[...], preferred_element_type=jnp.float32) + b_ref[1:2, :]
    r = jax.nn.sigmoid(gi[:, :C] + gh[:, :C])
    z = jax.nn.sigmoid(gi[:, C:2 * C] + gh[:, C:2 * C])
    n = jnp.tanh(gi[:, 2 * C:] + r * gh[:, 2 * C:])
    return (1.0 - z) * n + z * h


def _gru_fused_body(p_ref, x_ref, wih_ref, whh_ref, b_ref, wn_ref,
                    x_out_ref, m_out_ref):
    xn = _gru_math(p_ref, x_ref, wih_ref, whh_ref, b_ref)
    x_out_ref[...] = xn
    m_out_ref[...] = jnp.dot(xn, wn_ref[...], preferred_element_type=jnp.float32)


def _gru_last_body(p_ref, x_ref, wih_ref, whh_ref, b_ref, x_out_ref):
    x_out_ref[...] = _gru_math(p_ref, x_ref, wih_ref, whh_ref, b_ref)


def _gru_specs():
    return [
        pl.BlockSpec((_BN, C), lambda i: (i, 0)),
        pl.BlockSpec((_BN, C), lambda i: (i, 0)),
        pl.BlockSpec((C, 3 * C), lambda i: (0, 0)),
        pl.BlockSpec((C, 3 * C), lambda i: (0, 0)),
        pl.BlockSpec((8, 3 * C), lambda i: (0, 0)),
    ]


def _gru_fused(agg, x, wih_t, whh_t, bias, w_next):
    return pl.pallas_call(
        _gru_fused_body,
        grid=(_GRID,),
        in_specs=_gru_specs() + [pl.BlockSpec((C, C), lambda i: (0, 0))],
        out_specs=[pl.BlockSpec((_BN, C), lambda i: (i, 0)),
                   pl.BlockSpec((_BN, C), lambda i: (i, 0))],
        out_shape=[jax.ShapeDtypeStruct((N_NODES, C), jnp.float32),
                   jax.ShapeDtypeStruct((N_NODES, C), jnp.float32)],
    )(agg, x, wih_t, whh_t, bias, w_next)


def _gru_last(agg, x, wih_t, whh_t, bias):
    return pl.pallas_call(
        _gru_last_body,
        grid=(_GRID,),
        in_specs=_gru_specs(),
        out_specs=pl.BlockSpec((_BN, C), lambda i: (i, 0)),
        out_shape=jax.ShapeDtypeStruct((N_NODES, C), jnp.float32),
    )(agg, x, wih_t, whh_t, bias)


def kernel(x, edge_index, weight, w_ih, w_hh, b_ih, b_hh):
    src = edge_index[0].astype(jnp.int32)
    dst = edge_index[1].astype(jnp.int32)
    n_edges = src.shape[0]
    pad = _EPAD - n_edges
    src_slabs = jnp.concatenate(
        [src, jnp.zeros((pad,), jnp.int32)]).reshape(_TOTAL_CHUNKS, _CHUNK)
    dst_slabs = jnp.concatenate(
        [dst, jnp.full((pad,), N_NODES, jnp.int32)]).reshape(_TOTAL_CHUNKS, _CHUNK)
    zeros = jnp.zeros((_ROWS_PER_TILE, _HC), jnp.float32)
    wih_t = w_ih.T
    whh_t = w_hh.T
    bias = jnp.zeros((8, 3 * C), jnp.float32).at[0].set(b_ih).at[1].set(b_hh)

    m = _mm(x, weight[0])
    for i in range(NUM_LAYERS):
        agg = _sc_edge_pass(m, src_slabs, dst_slabs, zeros)
        if i + 1 < NUM_LAYERS:
            x, m = _gru_fused(agg, x, wih_t, whh_t, bias, weight[i + 1])
        else:
            x = _gru_last(agg, x, wih_t, whh_t, bias)
    return x


# speedup vs baseline: 3.4418x; 1.2435x over previous
import functools

import jax
import jax.numpy as jnp
from jax import lax
from jax.experimental import pallas as pl
from jax.experimental.pallas import tpu as pltpu
from jax.experimental.pallas import tpu_sc as plsc

N_NODES = 10000
C = 128
NUM_LAYERS = 3

_NC = 2
_NS = 16
_CHUNK = 128
_HC = C // _NC
_TOTAL_CHUNKS = 2560
_CHUNKS_PER_TILE = _TOTAL_CHUNKS // _NS
_EPAD = _TOTAL_CHUNKS * _CHUNK
_M_ROWS_PER_TILE = 625
_ROWS_PER_TILE = 640
_ACC_ROWS = _NS * _ROWS_PER_TILE

_BN = 1000
_GRID = N_NODES // _BN


_NBUF = 4
_SEG = 16
_NSEG = _CHUNKS_PER_TILE // _SEG
_SEG_ROUNDS = _SEG // _NBUF


@functools.cache
def _sc_edge_pass_kernel():
    mesh = plsc.VectorSubcoreMesh(core_axis_name="c", subcore_axis_name="s")

    @functools.partial(
        pl.kernel,
        out_type=jax.ShapeDtypeStruct((_ACC_ROWS, C), jnp.float32),
        mesh=mesh,
        compiler_params=pltpu.CompilerParams(use_tc_tiling_on_sc=False),
        scratch_types=[
            pltpu.VMEM((_SEG, _CHUNK), jnp.int32),
            pltpu.VMEM((_SEG, _CHUNK), jnp.int32),
        ]
        + [pltpu.VMEM((_CHUNK, _HC), jnp.float32) for _ in range(_NBUF)]
        + [pltpu.VMEM_SHARED((N_NODES, _HC), jnp.float32)]
        + [pltpu.VMEM_SHARED((_ACC_ROWS, _HC), jnp.float32)]
        + [pltpu.SemaphoreType.DMA for _ in range(2 * _NBUF)],
    )
    def sc_edge_pass(m_hbm, src_hbm, dst_hbm, zeros_hbm, out_hbm,
                     src_v, dst_v, *rest):
        msgs = rest[:_NBUF]
        m_sh = rest[_NBUF]
        acc_sh = rest[_NBUF + 1]
        sem_g = rest[_NBUF + 2:_NBUF + 2 + _NBUF]
        sem_s = rest[_NBUF + 2 + _NBUF:]
        c = lax.axis_index("c")
        s = lax.axis_index("s")
        pltpu.sync_copy(
            m_hbm.at[pl.ds(s * _M_ROWS_PER_TILE, _M_ROWS_PER_TILE),
                     pl.ds(c * _HC, _HC)],
            m_sh.at[pl.ds(s * _M_ROWS_PER_TILE, _M_ROWS_PER_TILE)])
        pltpu.sync_copy(zeros_hbm, acc_sh.at[pl.ds(s * _ROWS_PER_TILE, _ROWS_PER_TILE)])
        plsc.subcore_barrier()

        base = s * _CHUNKS_PER_TILE

        def gth(j, b):
            pltpu.async_copy(m_sh.at[src_v.at[j]], msgs[b], sem_g[b])

        def wait_gth(j, b):
            pltpu.make_async_copy(m_sh.at[src_v.at[j]], msgs[b], sem_g[b]).wait()

        def sct(j, b):
            pltpu.async_copy(msgs[b], acc_sh.at[dst_v.at[j]], sem_s[b], add=True)

        def wait_sct(j, b):
            pltpu.make_async_copy(msgs[b], acc_sh.at[dst_v.at[j]], sem_s[b]).wait()

        def seg_body(seg, carry):
            sb = base + seg * _SEG
            pltpu.sync_copy(src_hbm.at[pl.ds(sb, _SEG)], src_v)
            pltpu.sync_copy(dst_hbm.at[pl.ds(sb, _SEG)], dst_v)
            gth(0, 0)
            gth(1, 1)
            for j in range(_SEG):
                b = j % _NBUF
                wait_gth(j, b)
                sct(j, b)
                if j + 2 < _SEG:
                    bn = (j + 2) % _NBUF
                    if j >= 2:
                        wait_sct(j - 2, bn)
                    gth(j + 2, bn)
            for j in range(_SEG - _NBUF, _SEG):
                wait_sct(j, j % _NBUF)
            return carry

        lax.fori_loop(0, _NSEG, seg_body, 0)

        plsc.subcore_barrier()
        pltpu.sync_copy(acc_sh.at[pl.ds(s * _ROWS_PER_TILE, _ROWS_PER_TILE)],
                        out_hbm.at[pl.ds(s * _ROWS_PER_TILE, _ROWS_PER_TILE),
                                   pl.ds(c * _HC, _HC)])

    return sc_edge_pass


def _sc_edge_pass(m, src_slabs, dst_slabs, zeros):
    return _sc_edge_pass_kernel()(m, src_slabs, dst_slabs, zeros)


def _mm_body(x_ref, w_ref, o_ref):
    o_ref[...] = jnp.dot(x_ref[...], w_ref[...],
                         preferred_element_type=jnp.float32)


def _mm(x, w):
    return pl.pallas_call(
        _mm_body,
        grid=(_GRID,),
        in_specs=[pl.BlockSpec((_BN, C), lambda i: (i, 0)),
                  pl.BlockSpec((C, C), lambda i: (0, 0))],
        out_specs=pl.BlockSpec((_BN, C), lambda i: (i, 0)),
        out_shape=jax.ShapeDtypeStruct((N_NODES, C), jnp.float32),
    )(x, w)


def _gru_math(p_ref, x_ref, wih_ref, whh_ref, b_ref):
    agg = p_ref[...]
    h = x_ref[...]
    gi = jnp.dot(agg, wih_ref[...], preferred_element_type=jnp.float32) + b_ref[0:1, :]
    gh = jnp.dot(h, whh_ref[...], preferred_element_type=jnp.float32) + b_ref[1:2, :]
    r = jax.nn.sigmoid(gi[:, :C] + gh[:, :C])
    z = jax.nn.sigmoid(gi[:, C:2 * C] + gh[:, C:2 * C])
    n = jnp.tanh(gi[:, 2 * C:] + r * gh[:, 2 * C:])
    return (1.0 - z) * n + z * h


def _gru_fused_body(p_ref, x_ref, wih_ref, whh_ref, b_ref, wn_ref,
                    x_out_ref, m_out_ref):
    xn = _gru_math(p_ref, x_ref, wih_ref, whh_ref, b_ref)
    x_out_ref[...] = xn
    m_out_ref[...] = jnp.dot(xn, wn_ref[...], preferred_element_type=jnp.float32)


def _gru_last_body(p_ref, x_ref, wih_ref, whh_ref, b_ref, x_out_ref):
    x_out_ref[...] = _gru_math(p_ref, x_ref, wih_ref, whh_ref, b_ref)


def _gru_specs():
    return [
        pl.BlockSpec((_BN, C), lambda i: (i, 0)),
        pl.BlockSpec((_BN, C), lambda i: (i, 0)),
        pl.BlockSpec((C, 3 * C), lambda i: (0, 0)),
        pl.BlockSpec((C, 3 * C), lambda i: (0, 0)),
        pl.BlockSpec((8, 3 * C), lambda i: (0, 0)),
    ]


def _gru_fused(agg, x, wih_t, whh_t, bias, w_next):
    return pl.pallas_call(
        _gru_fused_body,
        grid=(_GRID,),
        in_specs=_gru_specs() + [pl.BlockSpec((C, C), lambda i: (0, 0))],
        out_specs=[pl.BlockSpec((_BN, C), lambda i: (i, 0)),
                   pl.BlockSpec((_BN, C), lambda i: (i, 0))],
        out_shape=[jax.ShapeDtypeStruct((N_NODES, C), jnp.float32),
                   jax.ShapeDtypeStruct((N_NODES, C), jnp.float32)],
    )(agg, x, wih_t, whh_t, bias, w_next)


def _gru_last(agg, x, wih_t, whh_t, bias):
    return pl.pallas_call(
        _gru_last_body,
        grid=(_GRID,),
        in_specs=_gru_specs(),
        out_specs=pl.BlockSpec((_BN, C), lambda i: (i, 0)),
        out_shape=jax.ShapeDtypeStruct((N_NODES, C), jnp.float32),
    )(agg, x, wih_t, whh_t, bias)


def kernel(x, edge_index, weight, w_ih, w_hh, b_ih, b_hh):
    src = edge_index[0].astype(jnp.int32)
    dst = edge_index[1].astype(jnp.int32)
    n_edges = src.shape[0]
    pad = _EPAD - n_edges
    src_slabs = jnp.concatenate(
        [src, jnp.zeros((pad,), jnp.int32)]).reshape(_TOTAL_CHUNKS, _CHUNK)
    dst_slabs = jnp.concatenate(
        [dst, jnp.full((pad,), N_NODES, jnp.int32)]).reshape(_TOTAL_CHUNKS, _CHUNK)
    zeros = jnp.zeros((_ROWS_PER_TILE, _HC), jnp.float32)
    wih_t = w_ih.T
    whh_t = w_hh.T
    bias = jnp.zeros((8, 3 * C), jnp.float32).at[0].set(b_ih).at[1].set(b_hh)

    m = _mm(x, weight[0])
    for i in range(NUM_LAYERS):
        agg = _sc_edge_pass(m, src_slabs, dst_slabs, zeros)
        if i + 1 < NUM_LAYERS:
            x, m = _gru_fused(agg, x, wih_t, whh_t, bias, weight[i + 1])
        else:
            x = _gru_last(agg, x, wih_t, whh_t, bias)
    return x


# speedup vs baseline: 3.6665x; 1.0653x over previous
import functools

import jax
import jax.numpy as jnp
from jax import lax
from jax.experimental import pallas as pl
from jax.experimental.pallas import tpu as pltpu
from jax.experimental.pallas import tpu_sc as plsc

N_NODES = 10000
C = 128
NUM_LAYERS = 3

_NC = 2
_NS = 16
_CHUNK = 128
_HC = C // _NC
_TOTAL_CHUNKS = 2560
_CHUNKS_PER_TILE = _TOTAL_CHUNKS // _NS
_EPAD = _TOTAL_CHUNKS * _CHUNK
_M_ROWS_PER_TILE = 625
_ROWS_PER_TILE = 640
_ACC_ROWS = _NS * _ROWS_PER_TILE

_BN = 1000
_GRID = N_NODES // _BN


_NBUF = 4
_SEG = 32
_NSEG = _CHUNKS_PER_TILE // _SEG
_SEG_ROUNDS = _SEG // _NBUF


@functools.cache
def _sc_edge_pass_kernel():
    mesh = plsc.VectorSubcoreMesh(core_axis_name="c", subcore_axis_name="s")

    @functools.partial(
        pl.kernel,
        out_type=jax.ShapeDtypeStruct((_ACC_ROWS, C), jnp.float32),
        mesh=mesh,
        compiler_params=pltpu.CompilerParams(use_tc_tiling_on_sc=False),
        scratch_types=[
            pltpu.VMEM((_SEG, _CHUNK), jnp.int32),
            pltpu.VMEM((_SEG, _CHUNK), jnp.int32),
        ]
        + [pltpu.VMEM((_CHUNK, _HC), jnp.float32) for _ in range(_NBUF)]
        + [pltpu.VMEM_SHARED((N_NODES, _HC), jnp.float32)]
        + [pltpu.VMEM_SHARED((_ACC_ROWS, _HC), jnp.float32)]
        + [pltpu.SemaphoreType.DMA for _ in range(2 * _NBUF)],
    )
    def sc_edge_pass(m_hbm, src_hbm, dst_hbm, zeros_hbm, out_hbm,
                     src_v, dst_v, *rest):
        msgs = rest[:_NBUF]
        m_sh = rest[_NBUF]
        acc_sh = rest[_NBUF + 1]
        sem_g = rest[_NBUF + 2:_NBUF + 2 + _NBUF]
        sem_s = rest[_NBUF + 2 + _NBUF:]
        c = lax.axis_index("c")
        s = lax.axis_index("s")
        pltpu.sync_copy(
            m_hbm.at[pl.ds(s * _M_ROWS_PER_TILE, _M_ROWS_PER_TILE),
                     pl.ds(c * _HC, _HC)],
            m_sh.at[pl.ds(s * _M_ROWS_PER_TILE, _M_ROWS_PER_TILE)])
        pltpu.sync_copy(zeros_hbm, acc_sh.at[pl.ds(s * _ROWS_PER_TILE, _ROWS_PER_TILE)])
        plsc.subcore_barrier()

        base = s * _CHUNKS_PER_TILE

        def gth(j, b):
            pltpu.async_copy(m_sh.at[src_v.at[j]], msgs[b], sem_g[b])

        def wait_gth(j, b):
            pltpu.make_async_copy(m_sh.at[src_v.at[j]], msgs[b], sem_g[b]).wait()

        def sct(j, b):
            pltpu.async_copy(msgs[b], acc_sh.at[dst_v.at[j]], sem_s[b], add=True)

        def wait_sct(j, b):
            pltpu.make_async_copy(msgs[b], acc_sh.at[dst_v.at[j]], sem_s[b]).wait()

        def seg_body(seg, carry):
            sb = base + seg * _SEG
            pltpu.sync_copy(src_hbm.at[pl.ds(sb, _SEG)], src_v)
            pltpu.sync_copy(dst_hbm.at[pl.ds(sb, _SEG)], dst_v)
            gth(0, 0)
            gth(1, 1)
            for j in range(_SEG):
                b = j % _NBUF
                wait_gth(j, b)
                sct(j, b)
                if j + 2 < _SEG:
                    bn = (j + 2) % _NBUF
                    if j >= 2:
                        wait_sct(j - 2, bn)
                    gth(j + 2, bn)
            for j in range(_SEG - _NBUF, _SEG):
                wait_sct(j, j % _NBUF)
            return carry

        lax.fori_loop(0, _NSEG, seg_body, 0)

        plsc.subcore_barrier()
        pltpu.sync_copy(acc_sh.at[pl.ds(s * _ROWS_PER_TILE, _ROWS_PER_TILE)],
                        out_hbm.at[pl.ds(s * _ROWS_PER_TILE, _ROWS_PER_TILE),
                                   pl.ds(c * _HC, _HC)])

    return sc_edge_pass


def _sc_edge_pass(m, src_slabs, dst_slabs, zeros):
    return _sc_edge_pass_kernel()(m, src_slabs, dst_slabs, zeros)


def _mm_body(x_ref, w_ref, o_ref):
    o_ref[...] = jnp.dot(x_ref[...], w_ref[...],
                         preferred_element_type=jnp.float32)


def _mm(x, w):
    return pl.pallas_call(
        _mm_body,
        grid=(_GRID,),
        in_specs=[pl.BlockSpec((_BN, C), lambda i: (i, 0)),
                  pl.BlockSpec((C, C), lambda i: (0, 0))],
        out_specs=pl.BlockSpec((_BN, C), lambda i: (i, 0)),
        out_shape=jax.ShapeDtypeStruct((N_NODES, C), jnp.float32),
    )(x, w)


def _gru_math(p_ref, x_ref, wih_ref, whh_ref, b_ref):
    agg = p_ref[...]
    h = x_ref[...]
    gi = jnp.dot(agg, wih_ref[...], preferred_element_type=jnp.float32) + b_ref[0:1, :]
    gh = jnp.dot(h, whh_ref[...], preferred_element_type=jnp.float32) + b_ref[1:2, :]
    r = jax.nn.sigmoid(gi[:, :C] + gh[:, :C])
    z = jax.nn.sigmoid(gi[:, C:2 * C] + gh[:, C:2 * C])
    n = jnp.tanh(gi[:, 2 * C:] + r * gh[:, 2 * C:])
    return (1.0 - z) * n + z * h


def _gru_fused_body(p_ref, x_ref, wih_ref, whh_ref, b_ref, wn_ref,
                    x_out_ref, m_out_ref):
    xn = _gru_math(p_ref, x_ref, wih_ref, whh_ref, b_ref)
    x_out_ref[...] = xn
    m_out_ref[...] = jnp.dot(xn, wn_ref[...], preferred_element_type=jnp.float32)


def _gru_last_body(p_ref, x_ref, wih_ref, whh_ref, b_ref, x_out_ref):
    x_out_ref[...] = _gru_math(p_ref, x_ref, wih_ref, whh_ref, b_ref)


def _gru_specs():
    return [
        pl.BlockSpec((_BN, C), lambda i: (i, 0)),
        pl.BlockSpec((_BN, C), lambda i: (i, 0)),
        pl.BlockSpec((C, 3 * C), lambda i: (0, 0)),
        pl.BlockSpec((C, 3 * C), lambda i: (0, 0)),
        pl.BlockSpec((8, 3 * C), lambda i: (0, 0)),
    ]


def _gru_fused(agg, x, wih_t, whh_t, bias, w_next):
    return pl.pallas_call(
        _gru_fused_body,
        grid=(_GRID,),
        in_specs=_gru_specs() + [pl.BlockSpec((C, C), lambda i: (0, 0))],
        out_specs=[pl.BlockSpec((_BN, C), lambda i: (i, 0)),
                   pl.BlockSpec((_BN, C), lambda i: (i, 0))],
        out_shape=[jax.ShapeDtypeStruct((N_NODES, C), jnp.float32),
                   jax.ShapeDtypeStruct((N_NODES, C), jnp.float32)],
    )(agg, x, wih_t, whh_t, bias, w_next)


def _gru_last(agg, x, wih_t, whh_t, bias):
    return pl.pallas_call(
        _gru_last_body,
        grid=(_GRID,),
        in_specs=_gru_specs(),
        out_specs=pl.BlockSpec((_BN, C), lambda i: (i, 0)),
        out_shape=jax.ShapeDtypeStruct((N_NODES, C), jnp.float32),
    )(agg, x, wih_t, whh_t, bias)


def kernel(x, edge_index, weight, w_ih, w_hh, b_ih, b_hh):
    src = edge_index[0].astype(jnp.int32)
    dst = edge_index[1].astype(jnp.int32)
    n_edges = src.shape[0]
    pad = _EPAD - n_edges
    src_slabs = jnp.concatenate(
        [src, jnp.zeros((pad,), jnp.int32)]).reshape(_TOTAL_CHUNKS, _CHUNK)
    dst_slabs = jnp.concatenate(
        [dst, jnp.full((pad,), N_NODES, jnp.int32)]).reshape(_TOTAL_CHUNKS, _CHUNK)
    zeros = jnp.zeros((_ROWS_PER_TILE, _HC), jnp.float32)
    wih_t = w_ih.T
    whh_t = w_hh.T
    bias = jnp.zeros((8, 3 * C), jnp.float32).at[0].set(b_ih).at[1].set(b_hh)

    m = _mm(x, weight[0])
    for i in range(NUM_LAYERS):
        agg = _sc_edge_pass(m, src_slabs, dst_slabs, zeros)
        if i + 1 < NUM_LAYERS:
            x, m = _gru_fused(agg, x, wih_t, whh_t, bias, weight[i + 1])
        else:
            x = _gru_last(agg, x, wih_t, whh_t, bias)
    return x


# speedup vs baseline: 3.7030x; 1.0099x over previous
import functools

import jax
import jax.numpy as jnp
from jax import lax
from jax.experimental import pallas as pl
from jax.experimental.pallas import tpu as pltpu
from jax.experimental.pallas import tpu_sc as plsc

N_NODES = 10000
C = 128
NUM_LAYERS = 3

_NC = 2
_NS = 16
_CHUNK = 128
_HC = C // _NC
_TOTAL_CHUNKS = 2560
_CHUNKS_PER_TILE = _TOTAL_CHUNKS // _NS
_EPAD = _TOTAL_CHUNKS * _CHUNK
_M_ROWS_PER_TILE = 625
_ROWS_PER_TILE = 640
_ACC_ROWS = _NS * _ROWS_PER_TILE

_BN = 1000
_GRID = N_NODES // _BN


_NBUF = 4
_SEG = 40
_NSEG = _CHUNKS_PER_TILE // _SEG
_SEG_ROUNDS = _SEG // _NBUF


@functools.cache
def _sc_edge_pass_kernel():
    mesh = plsc.VectorSubcoreMesh(core_axis_name="c", subcore_axis_name="s")

    @functools.partial(
        pl.kernel,
        out_type=jax.ShapeDtypeStruct((_ACC_ROWS, C), jnp.float32),
        mesh=mesh,
        compiler_params=pltpu.CompilerParams(use_tc_tiling_on_sc=False),
        scratch_types=[
            pltpu.VMEM((_SEG, _CHUNK), jnp.int32),
            pltpu.VMEM((_SEG, _CHUNK), jnp.int32),
        ]
        + [pltpu.VMEM((_CHUNK, _HC), jnp.float32) for _ in range(_NBUF)]
        + [pltpu.VMEM_SHARED((N_NODES, _HC), jnp.float32)]
        + [pltpu.VMEM_SHARED((_ACC_ROWS, _HC), jnp.float32)]
        + [pltpu.SemaphoreType.DMA for _ in range(2 * _NBUF)],
    )
    def sc_edge_pass(m_hbm, src_hbm, dst_hbm, zeros_hbm, out_hbm,
                     src_v, dst_v, *rest):
        msgs = rest[:_NBUF]
        m_sh = rest[_NBUF]
        acc_sh = rest[_NBUF + 1]
        sem_g = rest[_NBUF + 2:_NBUF + 2 + _NBUF]
        sem_s = rest[_NBUF + 2 + _NBUF:]
        c = lax.axis_index("c")
        s = lax.axis_index("s")
        pltpu.sync_copy(
            m_hbm.at[pl.ds(s * _M_ROWS_PER_TILE, _M_ROWS_PER_TILE),
                     pl.ds(c * _HC, _HC)],
            m_sh.at[pl.ds(s * _M_ROWS_PER_TILE, _M_ROWS_PER_TILE)])
        pltpu.sync_copy(zeros_hbm, acc_sh.at[pl.ds(s * _ROWS_PER_TILE, _ROWS_PER_TILE)])
        plsc.subcore_barrier()

        base = s * _CHUNKS_PER_TILE

        def gth(j, b):
            pltpu.async_copy(m_sh.at[src_v.at[j]], msgs[b], sem_g[b])

        def wait_gth(j, b):
            pltpu.make_async_copy(m_sh.at[src_v.at[j]], msgs[b], sem_g[b]).wait()

        def sct(j, b):
            pltpu.async_copy(msgs[b], acc_sh.at[dst_v.at[j]], sem_s[b], add=True)

        def wait_sct(j, b):
            pltpu.make_async_copy(msgs[b], acc_sh.at[dst_v.at[j]], sem_s[b]).wait()

        def seg_body(seg, carry):
            sb = base + seg * _SEG
            pltpu.sync_copy(src_hbm.at[pl.ds(sb, _SEG)], src_v)
            pltpu.sync_copy(dst_hbm.at[pl.ds(sb, _SEG)], dst_v)
            gth(0, 0)
            gth(1, 1)
            for j in range(_SEG):
                b = j % _NBUF
                wait_gth(j, b)
                sct(j, b)
                if j + 2 < _SEG:
                    bn = (j + 2) % _NBUF
                    if j >= 2:
                        wait_sct(j - 2, bn)
                    gth(j + 2, bn)
            for j in range(_SEG - _NBUF, _SEG):
                wait_sct(j, j % _NBUF)
            return carry

        lax.fori_loop(0, _NSEG, seg_body, 0)

        plsc.subcore_barrier()
        pltpu.sync_copy(acc_sh.at[pl.ds(s * _ROWS_PER_TILE, _ROWS_PER_TILE)],
                        out_hbm.at[pl.ds(s * _ROWS_PER_TILE, _ROWS_PER_TILE),
                                   pl.ds(c * _HC, _HC)])

    return sc_edge_pass


def _sc_edge_pass(m, src_slabs, dst_slabs, zeros):
    return _sc_edge_pass_kernel()(m, src_slabs, dst_slabs, zeros)


def _mm_body(x_ref, w_ref, o_ref):
    o_ref[...] = jnp.dot(x_ref[...], w_ref[...],
                         preferred_element_type=jnp.float32)


def _mm(x, w):
    return pl.pallas_call(
        _mm_body,
        grid=(_GRID,),
        in_specs=[pl.BlockSpec((_BN, C), lambda i: (i, 0)),
                  pl.BlockSpec((C, C), lambda i: (0, 0))],
        out_specs=pl.BlockSpec((_BN, C), lambda i: (i, 0)),
        out_shape=jax.ShapeDtypeStruct((N_NODES, C), jnp.float32),
    )(x, w)


def _gru_math(p_ref, x_ref, wih_ref, whh_ref, b_ref):
    agg = p_ref[...]
    h = x_ref[...]
    gi = jnp.dot(agg, wih_ref[...], preferred_element_type=jnp.float32) + b_ref[0:1, :]
    gh = jnp.dot(h, whh_ref[...], preferred_element_type=jnp.float32) + b_ref[1:2, :]
    r = jax.nn.sigmoid(gi[:, :C] + gh[:, :C])
    z = jax.nn.sigmoid(gi[:, C:2 * C] + gh[:, C:2 * C])
    n = jnp.tanh(gi[:, 2 * C:] + r * gh[:, 2 * C:])
    return (1.0 - z) * n + z * h


def _gru_fused_body(p_ref, x_ref, wih_ref, whh_ref, b_ref, wn_ref,
                    x_out_ref, m_out_ref):
    xn = _gru_math(p_ref, x_ref, wih_ref, whh_ref, b_ref)
    x_out_ref[...] = xn
    m_out_ref[...] = jnp.dot(xn, wn_ref[...], preferred_element_type=jnp.float32)


def _gru_last_body(p_ref, x_ref, wih_ref, whh_ref, b_ref, x_out_ref):
    x_out_ref[...] = _gru_math(p_ref, x_ref, wih_ref, whh_ref, b_ref)


def _gru_specs():
    return [
        pl.BlockSpec((_BN, C), lambda i: (i, 0)),
        pl.BlockSpec((_BN, C), lambda i: (i, 0)),
        pl.BlockSpec((C, 3 * C), lambda i: (0, 0)),
        pl.BlockSpec((C, 3 * C), lambda i: (0, 0)),
        pl.BlockSpec((8, 3 * C), lambda i: (0, 0)),
    ]


def _gru_fused(agg, x, wih_t, whh_t, bias, w_next):
    return pl.pallas_call(
        _gru_fused_body,
        grid=(_GRID,),
        in_specs=_gru_specs() + [pl.BlockSpec((C, C), lambda i: (0, 0))],
        out_specs=[pl.BlockSpec((_BN, C), lambda i: (i, 0)),
                   pl.BlockSpec((_BN, C), lambda i: (i, 0))],
        out_shape=[jax.ShapeDtypeStruct((N_NODES, C), jnp.float32),
                   jax.ShapeDtypeStruct((N_NODES, C), jnp.float32)],
    )(agg, x, wih_t, whh_t, bias, w_next)


def _gru_last(agg, x, wih_t, whh_t, bias):
    return pl.pallas_call(
        _gru_last_body,
        grid=(_GRID,),
        in_specs=_gru_specs(),
        out_specs=pl.BlockSpec((_BN, C), lambda i: (i, 0)),
        out_shape=jax.ShapeDtypeStruct((N_NODES, C), jnp.float32),
    )(agg, x, wih_t, whh_t, bias)


def kernel(x, edge_index, weight, w_ih, w_hh, b_ih, b_hh):
    src = edge_index[0].astype(jnp.int32)
    dst = edge_index[1].astype(jnp.int32)
    n_edges = src.shape[0]
    pad = _EPAD - n_edges
    src_slabs = jnp.concatenate(
        [src, jnp.zeros((pad,), jnp.int32)]).reshape(_TOTAL_CHUNKS, _CHUNK)
    dst_slabs = jnp.concatenate(
        [dst, jnp.full((pad,), N_NODES, jnp.int32)]).reshape(_TOTAL_CHUNKS, _CHUNK)
    zeros = jnp.zeros((_ROWS_PER_TILE, _HC), jnp.float32)
    wih_t = w_ih.T
    whh_t = w_hh.T
    bias = jnp.zeros((8, 3 * C), jnp.float32).at[0].set(b_ih).at[1].set(b_hh)

    m = _mm(x, weight[0])
    for i in range(NUM_LAYERS):
        agg = _sc_edge_pass(m, src_slabs, dst_slabs, zeros)
        if i + 1 < NUM_LAYERS:
            x, m = _gru_fused(agg, x, wih_t, whh_t, bias, weight[i + 1])
        else:
            x = _gru_last(agg, x, wih_t, whh_t, bias)
    return x


# speedup vs baseline: 3.7584x; 1.0150x over previous
import functools

import jax
import jax.numpy as jnp
from jax import lax
from jax.experimental import pallas as pl
from jax.experimental.pallas import tpu as pltpu
from jax.experimental.pallas import tpu_sc as plsc

N_NODES = 10000
C = 128
NUM_LAYERS = 3

_NC = 2
_NS = 16
_CHUNK = 128
_HC = C // _NC
_TOTAL_CHUNKS = 2560
_CHUNKS_PER_TILE = _TOTAL_CHUNKS // _NS
_EPAD = _TOTAL_CHUNKS * _CHUNK
_M_ROWS_PER_TILE = 625
_ROWS_PER_TILE = 640
_ACC_ROWS = _NS * _ROWS_PER_TILE

_BN = 1000
_GRID = N_NODES // _BN


_NBUF = 4
_SEG = 40
_NSEG = _CHUNKS_PER_TILE // _SEG
_SEG_ROUNDS = _SEG // _NBUF


@functools.cache
def _sc_edge_pass_kernel():
    mesh = plsc.VectorSubcoreMesh(core_axis_name="c", subcore_axis_name="s")

    @functools.partial(
        pl.kernel,
        out_type=jax.ShapeDtypeStruct((_ACC_ROWS, C), jnp.float32),
        mesh=mesh,
        compiler_params=pltpu.CompilerParams(use_tc_tiling_on_sc=False),
        scratch_types=[
            pltpu.VMEM((_SEG, _CHUNK), jnp.int32),
            pltpu.VMEM((_SEG, _CHUNK), jnp.int32),
        ]
        + [pltpu.VMEM((_CHUNK, _HC), jnp.float32) for _ in range(_NBUF)]
        + [pltpu.VMEM_SHARED((N_NODES, _HC), jnp.float32)]
        + [pltpu.VMEM_SHARED((_ACC_ROWS, _HC), jnp.float32)]
        + [pltpu.SemaphoreType.DMA for _ in range(2 * _NBUF)],
    )
    def sc_edge_pass(m_hbm, src_hbm, dst_hbm, zeros_hbm, out_hbm,
                     src_v, dst_v, *rest):
        msgs = rest[:_NBUF]
        m_sh = rest[_NBUF]
        acc_sh = rest[_NBUF + 1]
        sem_g = rest[_NBUF + 2:_NBUF + 2 + _NBUF]
        sem_s = rest[_NBUF + 2 + _NBUF:]
        c = lax.axis_index("c")
        s = lax.axis_index("s")
        pltpu.sync_copy(
            m_hbm.at[pl.ds(s * _M_ROWS_PER_TILE, _M_ROWS_PER_TILE),
                     pl.ds(c * _HC, _HC)],
            m_sh.at[pl.ds(s * _M_ROWS_PER_TILE, _M_ROWS_PER_TILE)])
        pltpu.sync_copy(zeros_hbm, acc_sh.at[pl.ds(s * _ROWS_PER_TILE, _ROWS_PER_TILE)])
        plsc.subcore_barrier()

        base = s * _CHUNKS_PER_TILE

        def gth(j, b):
            pltpu.async_copy(m_sh.at[src_v.at[j]], msgs[b], sem_g[b])

        def wait_gth(j, b):
            pltpu.make_async_copy(m_sh.at[src_v.at[j]], msgs[b], sem_g[b]).wait()

        def sct(j, b):
            pltpu.async_copy(msgs[b], acc_sh.at[dst_v.at[j]], sem_s[b], add=True)

        def wait_sct(j, b):
            pltpu.make_async_copy(msgs[b], acc_sh.at[dst_v.at[j]], sem_s[b]).wait()

        def seg_body(seg, carry):
            sb = base + seg * _SEG
            pltpu.async_copy(src_hbm.at[pl.ds(sb, _SEG)], src_v, sem_g[0])
            pltpu.async_copy(dst_hbm.at[pl.ds(sb, _SEG)], dst_v, sem_g[1])
            pltpu.make_async_copy(src_hbm.at[pl.ds(sb, _SEG)], src_v, sem_g[0]).wait()
            pltpu.make_async_copy(dst_hbm.at[pl.ds(sb, _SEG)], dst_v, sem_g[1]).wait()
            gth(0, 0)
            gth(1, 1)
            for j in range(_SEG):
                b = j % _NBUF
                wait_gth(j, b)
                sct(j, b)
                if j + 2 < _SEG:
                    bn = (j + 2) % _NBUF
                    if j >= 2:
                        wait_sct(j - 2, bn)
                    gth(j + 2, bn)
            for j in range(_SEG - _NBUF, _SEG):
                wait_sct(j, j % _NBUF)
            return carry

        lax.fori_loop(0, _NSEG, seg_body, 0)

        plsc.subcore_barrier()
        pltpu.sync_copy(acc_sh.at[pl.ds(s * _ROWS_PER_TILE, _ROWS_PER_TILE)],
                        out_hbm.at[pl.ds(s * _ROWS_PER_TILE, _ROWS_PER_TILE),
                                   pl.ds(c * _HC, _HC)])

    return sc_edge_pass


def _sc_edge_pass(m, src_slabs, dst_slabs, zeros):
    return _sc_edge_pass_kernel()(m, src_slabs, dst_slabs, zeros)


def _mm_body(x_ref, w_ref, o_ref):
    o_ref[...] = jnp.dot(x_ref[...], w_ref[...],
                         preferred_element_type=jnp.float32)


def _mm(x, w):
    return pl.pallas_call(
        _mm_body,
        grid=(_GRID,),
        in_specs=[pl.BlockSpec((_BN, C), lambda i: (i, 0)),
                  pl.BlockSpec((C, C), lambda i: (0, 0))],
        out_specs=pl.BlockSpec((_BN, C), lambda i: (i, 0)),
        out_shape=jax.ShapeDtypeStruct((N_NODES, C), jnp.float32),
    )(x, w)


def _gru_math(p_ref, x_ref, wih_ref, whh_ref, b_ref):
    agg = p_ref[...]
    h = x_ref[...]
    gi = jnp.dot(agg, wih_ref[...], preferred_element_type=jnp.float32) + b_ref[0:1, :]
    gh = jnp.dot(h, whh_ref[...], preferred_element_type=jnp.float32) + b_ref[1:2, :]
    r = jax.nn.sigmoid(gi[:, :C] + gh[:, :C])
    z = jax.nn.sigmoid(gi[:, C:2 * C] + gh[:, C:2 * C])
    n = jnp.tanh(gi[:, 2 * C:] + r * gh[:, 2 * C:])
    return (1.0 - z) * n + z * h


def _gru_fused_body(p_ref, x_ref, wih_ref, whh_ref, b_ref, wn_ref,
                    x_out_ref, m_out_ref):
    xn = _gru_math(p_ref, x_ref, wih_ref, whh_ref, b_ref)
    x_out_ref[...] = xn
    m_out_ref[...] = jnp.dot(xn, wn_ref[...], preferred_element_type=jnp.float32)


def _gru_last_body(p_ref, x_ref, wih_ref, whh_ref, b_ref, x_out_ref):
    x_out_ref[...] = _gru_math(p_ref, x_ref, wih_ref, whh_ref, b_ref)


def _gru_specs():
    return [
        pl.BlockSpec((_BN, C), lambda i: (i, 0)),
        pl.BlockSpec((_BN, C), lambda i: (i, 0)),
        pl.BlockSpec((C, 3 * C), lambda i: (0, 0)),
        pl.BlockSpec((C, 3 * C), lambda i: (0, 0)),
        pl.BlockSpec((8, 3 * C), lambda i: (0, 0)),
    ]


def _gru_fused(agg, x, wih_t, whh_t, bias, w_next):
    return pl.pallas_call(
        _gru_fused_body,
        grid=(_GRID,),
        in_specs=_gru_specs() + [pl.BlockSpec((C, C), lambda i: (0, 0))],
        out_specs=[pl.BlockSpec((_BN, C), lambda i: (i, 0)),
                   pl.BlockSpec((_BN, C), lambda i: (i, 0))],
        out_shape=[jax.ShapeDtypeStruct((N_NODES, C), jnp.float32),
                   jax.ShapeDtypeStruct((N_NODES, C), jnp.float32)],
    )(agg, x, wih_t, whh_t, bias, w_next)


def _gru_last(agg, x, wih_t, whh_t, bias):
    return pl.pallas_call(
        _gru_last_body,
        grid=(_GRID,),
        in_specs=_gru_specs(),
        out_specs=pl.BlockSpec((_BN, C), lambda i: (i, 0)),
        out_shape=jax.ShapeDtypeStruct((N_NODES, C), jnp.float32),
    )(agg, x, wih_t, whh_t, bias)


def kernel(x, edge_index, weight, w_ih, w_hh, b_ih, b_hh):
    src = edge_index[0].astype(jnp.int32)
    dst = edge_index[1].astype(jnp.int32)
    n_edges = src.shape[0]
    pad = _EPAD - n_edges
    src_slabs = jnp.concatenate(
        [src, jnp.zeros((pad,), jnp.int32)]).reshape(_TOTAL_CHUNKS, _CHUNK)
    dst_slabs = jnp.concatenate(
        [dst, jnp.full((pad,), N_NODES, jnp.int32)]).reshape(_TOTAL_CHUNKS, _CHUNK)
    zeros = jnp.zeros((_ROWS_PER_TILE, _HC), jnp.float32)
    wih_t = w_ih.T
    whh_t = w_hh.T
    bias = jnp.zeros((8, 3 * C), jnp.float32).at[0].set(b_ih).at[1].set(b_hh)

    m = _mm(x, weight[0])
    for i in range(NUM_LAYERS):
        agg = _sc_edge_pass(m, src_slabs, dst_slabs, zeros)
        if i + 1 < NUM_LAYERS:
            x, m = _gru_fused(agg, x, wih_t, whh_t, bias, weight[i + 1])
        else:
            x = _gru_last(agg, x, wih_t, whh_t, bias)
    return x
